# baseline (speedup 1.0000x reference)
import functools

import jax
import jax.numpy as jnp
from jax import lax
from jax.experimental import pallas as pl
from jax.experimental.pallas import tpu as pltpu
from jax.experimental.pallas import tpu_sc as plsc

D = 64
BATCH = 4096
HIST = 200
NC = 2
NS = 16
NW = NC * NS
BB = BATCH // NW
N_OUTER = HIST // 2
L = 16

_mesh = plsc.VectorSubcoreMesh(core_axis_name="c", subcore_axis_name="s")


@functools.partial(
    pl.kernel,
    mesh=_mesh,
    out_type=jax.ShapeDtypeStruct((HIST, D, BATCH), jnp.float32),
    scratch_types=[
        pltpu.VMEM((HIST, BB), jnp.int32),
        pltpu.VMEM((BB, D), jnp.float32),
        pltpu.VMEM((BB, D), jnp.float32),
        pltpu.VMEM((D, BB), jnp.float32),
        pltpu.VMEM((D, BB), jnp.float32),
        pltpu.SemaphoreType.DMA,
        pltpu.SemaphoreType.DMA,
        pltpu.SemaphoreType.DMA,
        pltpu.SemaphoreType.DMA,
    ],
    compiler_params=pltpu.CompilerParams(use_tc_tiling_on_sc=False,
                                         needs_layout_passes=False),
)
def _gather_kernel(idx_hbm, table_hbm, out_hbm,
                   idxb, x0, x1, t0, t1, sg0, sg1, ss0, ss1):
    wid = lax.axis_index("s") * NC + lax.axis_index("c")
    b0 = wid * BB

    pltpu.sync_copy(idx_hbm.at[pl.ds(0, HIST), pl.ds(b0, BB)], idxb)

    def start_gather(h, x, sem):
        pltpu.async_copy(table_hbm.at[idxb.at[h]], x, sem)

    def wait_gather(x, sem):
        pltpu.make_async_copy(table_hbm.at[idxb.at[0]], x, sem).wait()

    def transpose(x, t):
        lanes = lax.iota(jnp.int32, L)
        def dbody(d, carry):
            for g in range(BB // L):
                v = plsc.load_gather(x, [g * L + lanes,
                                         jnp.full((L,), d, jnp.int32)])
                t[d, pl.ds(g * L, L)] = v
            return carry
        lax.fori_loop(0, D, dbody, 0)

    def start_store(h, t, sem):
        pltpu.async_copy(t, out_hbm.at[h, pl.ds(0, D), pl.ds(b0, BB)], sem)

    def wait_store(t, sem):
        pltpu.make_async_copy(t, out_hbm.at[0, pl.ds(0, D), pl.ds(b0, BB)],
                              sem).wait()

    start_gather(0, x0, sg0)
    start_gather(1, x1, sg1)
    wait_gather(x0, sg0)
    transpose(x0, t0)
    start_store(0, t0, ss0)
    start_gather(2, x0, sg0)
    wait_gather(x1, sg1)
    transpose(x1, t1)
    start_store(1, t1, ss1)

    def body(i, carry):
        h = 2 * i
        start_gather(h + 1, x1, sg1)
        wait_gather(x0, sg0)
        wait_store(t0, ss0)
        transpose(x0, t0)
        start_store(h, t0, ss0)
        start_gather(h + 2, x0, sg0)
        wait_gather(x1, sg1)
        wait_store(t1, ss1)
        transpose(x1, t1)
        start_store(h + 1, t1, ss1)
        return carry

    lax.fori_loop(1, N_OUTER - 1, body, 0)

    h = HIST - 2
    start_gather(h + 1, x1, sg1)
    wait_gather(x0, sg0)
    wait_store(t0, ss0)
    transpose(x0, t0)
    start_store(h, t0, ss0)
    wait_gather(x1, sg1)
    wait_store(t1, ss1)
    transpose(x1, t1)
    start_store(h + 1, t1, ss1)
    wait_store(t0, ss0)
    wait_store(t1, ss1)


def kernel(inputs, word_embedding_matrix):
    idx_t = inputs.T.astype(jnp.int32)
    out_t = _gather_kernel(idx_t, word_embedding_matrix)
    return jnp.transpose(out_t, (2, 0, 1))


# speedup vs baseline: 3.2189x; 3.2189x over previous
import functools

import jax
import jax.numpy as jnp
from jax import lax
from jax.experimental import pallas as pl
from jax.experimental.pallas import tpu as pltpu
from jax.experimental.pallas import tpu_sc as plsc

D = 64
BATCH = 4096
HIST = 200
NC = 2
NS = 16
NW = NC * NS
BB = BATCH // NW
N_OUTER = HIST // 2
L = 16
TS = 129
UNROLL = 4

_mesh = plsc.VectorSubcoreMesh(core_axis_name="c", subcore_axis_name="s")


@functools.partial(
    pl.kernel,
    mesh=_mesh,
    out_type=jax.ShapeDtypeStruct((HIST, D // 8, BATCH // BB, 8, BB),
                                  jnp.float32),
    scratch_types=[
        pltpu.VMEM((HIST, BB), jnp.int32),
        pltpu.VMEM((BB, D), jnp.float32),
        pltpu.VMEM((BB, D), jnp.float32),
        pltpu.VMEM((D, TS), jnp.float32),
        pltpu.VMEM((D, TS), jnp.float32),
        pltpu.SemaphoreType.DMA,
        pltpu.SemaphoreType.DMA,
        pltpu.SemaphoreType.DMA,
        pltpu.SemaphoreType.DMA,
    ],
    compiler_params=pltpu.CompilerParams(use_tc_tiling_on_sc=False,
                                         needs_layout_passes=False),
)
def _gather_kernel(idx_hbm, table_hbm, out_hbm,
                   idxb, x0, x1, t0, t1, sg0, sg1, ss0, ss1):
    wid = lax.axis_index("s") * NC + lax.axis_index("c")
    tb = wid

    pltpu.sync_copy(idx_hbm.at[pl.ds(0, HIST), pl.ds(tb * BB, BB)], idxb)

    def start_gather(h, x, sem):
        pltpu.async_copy(table_hbm.at[idxb.at[h]], x, sem)

    def wait_gather(x, sem):
        pltpu.make_async_copy(table_hbm.at[idxb.at[0]], x, sem).wait()

    def transpose(x, t):
        lanes = lax.iota(jnp.int32, L)
        def bbody(r, carry):
            for u in range(UNROLL):
                b = r * UNROLL + u
                for c in range(D // L):
                    v = x[b, pl.ds(c * L, L)]
                    plsc.store_scatter(t, [c * L + lanes,
                                           jnp.full((L,), b, jnp.int32)], v)
            return carry
        lax.fori_loop(0, BB // UNROLL, bbody, 0)

    def start_store(h, t, sem):
        for td in range(D // 8):
            pltpu.async_copy(t.at[pl.ds(td * 8, 8), pl.ds(0, BB)],
                             out_hbm.at[h, td, tb], sem)

    def wait_store(t, sem):
        for td in range(D // 8):
            pltpu.make_async_copy(t.at[pl.ds(td * 8, 8), pl.ds(0, BB)],
                                  out_hbm.at[0, td, tb], sem).wait()

    start_gather(0, x0, sg0)
    start_gather(1, x1, sg1)
    wait_gather(x0, sg0)
    transpose(x0, t0)
    start_store(0, t0, ss0)
    start_gather(2, x0, sg0)
    wait_gather(x1, sg1)
    transpose(x1, t1)
    start_store(1, t1, ss1)

    def body(i, carry):
        h = 2 * i
        start_gather(h + 1, x1, sg1)
        wait_gather(x0, sg0)
        wait_store(t0, ss0)
        transpose(x0, t0)
        start_store(h, t0, ss0)
        start_gather(h + 2, x0, sg0)
        wait_gather(x1, sg1)
        wait_store(t1, ss1)
        transpose(x1, t1)
        start_store(h + 1, t1, ss1)
        return carry

    lax.fori_loop(1, N_OUTER - 1, body, 0)

    h = HIST - 2
    start_gather(h + 1, x1, sg1)
    wait_gather(x0, sg0)
    wait_store(t0, ss0)
    transpose(x0, t0)
    start_store(h, t0, ss0)
    wait_gather(x1, sg1)
    wait_store(t1, ss1)
    transpose(x1, t1)
    start_store(h + 1, t1, ss1)
    wait_store(t0, ss0)
    wait_store(t1, ss1)


def kernel(inputs, word_embedding_matrix):
    idx_t = inputs.T.astype(jnp.int32)
    out5 = _gather_kernel(idx_t, word_embedding_matrix)
    return jnp.transpose(out5, (2, 4, 0, 1, 3)).reshape(BATCH, HIST, D)


# speedup vs baseline: 3.7500x; 1.1650x over previous
import functools

import jax
import jax.numpy as jnp
from jax import lax
from jax.experimental import pallas as pl
from jax.experimental.pallas import tpu as pltpu
from jax.experimental.pallas import tpu_sc as plsc

D = 64
DP = 128
BATCH = 4096
HIST = 200
NC = 2
NS = 16
NW = NC * NS
ROWS_PER_W = BATCH // NW
N_OUTER = ROWS_PER_W // 2

_mesh = plsc.VectorSubcoreMesh(core_axis_name="c", subcore_axis_name="s")


@functools.partial(
    pl.kernel,
    mesh=_mesh,
    out_type=jax.ShapeDtypeStruct((BATCH, HIST, DP), jnp.float32),
    scratch_types=[
        pltpu.VMEM((ROWS_PER_W, HIST), jnp.int32),
        pltpu.VMEM((HIST, D), jnp.float32),
        pltpu.VMEM((HIST, D), jnp.float32),
        pltpu.SemaphoreType.DMA,
        pltpu.SemaphoreType.DMA,
        pltpu.SemaphoreType.DMA,
        pltpu.SemaphoreType.DMA,
    ],
    compiler_params=pltpu.CompilerParams(use_tc_tiling_on_sc=False),
)
def _gather_kernel(idx_hbm, table_hbm, out_hbm,
                   idx_all, rows0, rows1, sg0, sg1, ss0, ss1):
    wid = lax.axis_index("s") * NC + lax.axis_index("c")
    base_w = wid * ROWS_PER_W

    pltpu.sync_copy(idx_hbm.at[pl.ds(base_w, ROWS_PER_W)], idx_all)

    def start_gather(j, rows, sem):
        pltpu.async_copy(table_hbm.at[idx_all.at[j]], rows, sem)

    def wait_gather(rows, sem):
        pltpu.make_async_copy(table_hbm.at[idx_all.at[0]], rows, sem).wait()

    def start_store(j, rows, sem):
        pltpu.async_copy(rows,
                         out_hbm.at[base_w + j, pl.ds(0, HIST), pl.ds(0, D)],
                         sem)

    def wait_store(rows, sem):
        pltpu.make_async_copy(rows,
                              out_hbm.at[0, pl.ds(0, HIST), pl.ds(0, D)],
                              sem).wait()

    start_gather(0, rows0, sg0)
    start_gather(1, rows1, sg1)
    wait_gather(rows0, sg0)
    start_store(0, rows0, ss0)
    wait_store(rows0, ss0)
    start_gather(2, rows0, sg0)
    wait_gather(rows1, sg1)
    start_store(1, rows1, ss1)

    def body(i, carry):
        j = 2 * i
        wait_store(rows1, ss1)
        start_gather(j + 1, rows1, sg1)
        wait_gather(rows0, sg0)
        start_store(j, rows0, ss0)
        wait_store(rows0, ss0)
        start_gather(j + 2, rows0, sg0)
        wait_gather(rows1, sg1)
        start_store(j + 1, rows1, ss1)
        return carry

    lax.fori_loop(1, N_OUTER - 1, body, 0)

    j = ROWS_PER_W - 2
    wait_store(rows1, ss1)
    start_gather(j + 1, rows1, sg1)
    wait_gather(rows0, sg0)
    start_store(j, rows0, ss0)
    wait_gather(rows1, sg1)
    start_store(j + 1, rows1, ss1)
    wait_store(rows0, ss0)
    wait_store(rows1, ss1)


def kernel(inputs, word_embedding_matrix):
    idx = inputs.astype(jnp.int32)
    out_p = _gather_kernel(idx, word_embedding_matrix)
    return out_p[..., :D]
